# baseline (device time: 159153 ns/iter reference)
import jax
import jax.numpy as jnp
from jax import lax
from jax.experimental import pallas as pl
from jax.experimental.pallas import tpu as pltpu

N_DEV = 32
DH = 64
WINDOW = 128


def _fused(x, Wq, K, V, Wo):
    B, Sq, E = x.shape
    h_per = Wq.shape[1] // DH
    bf16 = jnp.bfloat16
    m = B * Sq
    n = E

    rs_sizes = [m >> (k + 1) for k in range(5)]
    rs_offs = [sum(rs_sizes[:k]) for k in range(5)]
    rs_total = sum(rs_sizes)

    def body(x_ref, wq_ref, k_ref, v_ref, wo_ref, out_ref,
             acc_ref, send_stage, rs_recv, ag_buf, send_sems, recv_sems):
        qi = lax.broadcasted_iota(jnp.int32, (Sq, Sq), 0)
        ki = lax.broadcasted_iota(jnp.int32, (Sq, Sq), 1)
        neg = jnp.float32(-1e9)
        band = jnp.abs(qi - ki) <= WINDOW
        hi = lax.rem(lax.axis_index("i"), 2) == 0
        for b in range(B):
            xb = x_ref[b, :, :].astype(bf16)
            acc = jnp.zeros((Sq, E), jnp.float32)
            for h in range(h_per):
                wq_h = wq_ref[:, h * DH:(h + 1) * DH].astype(bf16)
                q = jnp.dot(xb, wq_h,
                            preferred_element_type=jnp.float32)
                q = (q * 0.125).astype(bf16)
                k = jnp.where(hi, k_ref[b, :, h, :], k_ref[b, :, h + 8, :])
                s = lax.dot_general(
                    q, k, (((1,), (1,)), ((), ())),
                    preferred_element_type=jnp.float32)
                s = jnp.where(band, s, neg)
                s = s - jnp.max(s, axis=1, keepdims=True)
                e = jnp.exp(s)
                w = e / jnp.sum(e, axis=1, keepdims=True)
                v = jnp.where(hi, v_ref[b, :, h, :], v_ref[b, :, h + 8, :])
                ctx = jnp.dot(w.astype(bf16), v,
                              preferred_element_type=jnp.float32)
                wo_h = wo_ref[h * DH:(h + 1) * DH, :].astype(bf16)
                acc = acc + jnp.dot(ctx.astype(bf16), wo_h,
                                    preferred_element_type=jnp.float32)
            acc_ref[b * Sq:(b + 1) * Sq, :] = acc

        i = lax.axis_index("i")
        z = i // 8
        p = lax.rem(i, 8)
        y = p // 2
        xb_ = jnp.bitwise_xor(lax.rem(p, 2), lax.rem(y, 2))

        def ring_idx(xx, yy, zz):
            pp = 2 * yy + jnp.bitwise_xor(xx, lax.rem(yy, 2))
            return 8 * zz + pp

        partners = [
            ring_idx(jnp.bitwise_xor(xb_, 1), y, z),
            ring_idx(xb_, jnp.bitwise_xor(y, 1), z),
            ring_idx(xb_, y, jnp.bitwise_xor(z, 1)),
            ring_idx(xb_, jnp.bitwise_xor(y, 2), z),
            ring_idx(xb_, y, jnp.bitwise_xor(z, 2)),
        ]
        bits = [
            lax.rem(xb_, 2),
            lax.rem(y, 2),
            lax.rem(z, 2),
            lax.rem(y // 2, 2),
            lax.rem(z // 2, 2),
        ]

        barrier_sem = pltpu.get_barrier_semaphore()
        for pk in partners:
            pl.semaphore_signal(
                barrier_sem, inc=1,
                device_id=(pk,), device_id_type=pl.DeviceIdType.MESH,
            )
        pl.semaphore_wait(barrier_sem, len(partners))

        seg_start = i * 0
        for k in range(5):
            half = rs_sizes[k]
            b = bits[k]
            send_off = seg_start + (1 - b) * half
            keep_off = seg_start + b * half
            send_stage[pl.ds(0, half), :] = acc_ref[
                pl.ds(send_off, half), :
            ].astype(bf16)
            rdma = pltpu.make_async_remote_copy(
                src_ref=send_stage.at[pl.ds(0, half)],
                dst_ref=rs_recv.at[pl.ds(rs_offs[k], half)],
                send_sem=send_sems.at[k],
                recv_sem=recv_sems.at[k],
                device_id=(partners[k],),
                device_id_type=pl.DeviceIdType.MESH,
            )
            rdma.start()
            rdma.wait()
            acc_ref[pl.ds(keep_off, half), :] = (
                acc_ref[pl.ds(keep_off, half), :]
                + rs_recv[pl.ds(rs_offs[k], half), :].astype(jnp.float32)
            )
            seg_start = keep_off

        own = m // N_DEV
        ag_buf[pl.ds(seg_start, own), :] = acc_ref[
            pl.ds(seg_start, own), :
        ].astype(bf16)
        own_start = seg_start
        for j in range(5):
            r = 4 - j
            b = bits[r]
            rdma = pltpu.make_async_remote_copy(
                src_ref=ag_buf.at[pl.ds(own_start, own)],
                dst_ref=ag_buf.at[pl.ds(own_start, own)],
                send_sem=send_sems.at[5 + j],
                recv_sem=recv_sems.at[5 + j],
                device_id=(partners[r],),
                device_id_type=pl.DeviceIdType.MESH,
            )
            rdma.start()
            rdma.wait()
            own_start = own_start - b * own
            own = own * 2

        out_ref[:, :] = ag_buf[:, :].astype(jnp.float32)

    return pl.pallas_call(
        body,
        out_shape=jax.ShapeDtypeStruct((m, n), jnp.float32),
        in_specs=[pl.BlockSpec(memory_space=pltpu.VMEM)] * 5,
        out_specs=pl.BlockSpec(memory_space=pltpu.VMEM),
        scratch_shapes=[
            pltpu.VMEM((m, n), jnp.float32),
            pltpu.VMEM((m // 2, n), bf16),
            pltpu.VMEM((rs_total, n), bf16),
            pltpu.VMEM((m, n), bf16),
            pltpu.SemaphoreType.DMA((10,)),
            pltpu.SemaphoreType.DMA((10,)),
        ],
        compiler_params=pltpu.CompilerParams(collective_id=0),
    )(x, Wq, K, V, Wo)


def kernel(x, Wq, K_ext, V_ext, Wo):
    my = lax.axis_index("i")
    B, Sq, E = x.shape
    h_per = Wq.shape[1] // DH

    off16 = (my // 2) * (2 * h_per)
    K = lax.dynamic_slice_in_dim(K_ext, off16, 2 * h_per, axis=2)
    V = lax.dynamic_slice_in_dim(V_ext, off16, 2 * h_per, axis=2)
    K = K.astype(jnp.bfloat16)
    V = V.astype(jnp.bfloat16)

    out = _fused(x, Wq, K, V, Wo)
    return out.reshape(B, Sq, E)


# device time: 154390 ns/iter; 1.0309x vs baseline; 1.0309x over previous
import jax
import jax.numpy as jnp
from jax import lax
from jax.experimental import pallas as pl
from jax.experimental.pallas import tpu as pltpu

N_DEV = 32
DH = 64
WINDOW = 128


def _fused(x, Wq, K, V, Wo):
    B, Sq, E = x.shape
    h_per = Wq.shape[1] // DH
    bf16 = jnp.bfloat16
    m = B * Sq
    n = E

    rs_sizes = [m >> (k + 1) for k in range(5)]
    rs_offs = [sum(rs_sizes[:k]) for k in range(5)]
    rs_total = sum(rs_sizes)

    def body(x_ref, wq_ref, k_ref, v_ref, wo_ref, out_ref,
             acc_ref, send_stage, rs_recv, ag_buf, send_sems, recv_sems):
        qi = lax.broadcasted_iota(jnp.int32, (Sq, Sq), 0)
        ki = lax.broadcasted_iota(jnp.int32, (Sq, Sq), 1)
        neg = jnp.float32(-1e9)
        band = jnp.abs(qi - ki) <= WINDOW
        for b in range(B):
            xb = x_ref[b, :, :].astype(bf16)
            acc = jnp.zeros((Sq, E), jnp.float32)
            for h in range(h_per):
                wq_h = wq_ref[:, h * DH:(h + 1) * DH].astype(bf16)
                q = jnp.dot(xb, wq_h,
                            preferred_element_type=jnp.float32)
                q = (q * 0.125).astype(bf16)
                k = k_ref[b, :, h, :]
                s = lax.dot_general(
                    q, k, (((1,), (1,)), ((), ())),
                    preferred_element_type=jnp.float32)
                s = jnp.where(band, s, neg)
                s = s - jnp.max(s, axis=1, keepdims=True)
                e = jnp.exp(s)
                w = e / jnp.sum(e, axis=1, keepdims=True)
                v = v_ref[b, :, h, :]
                ctx = jnp.dot(w.astype(bf16), v,
                              preferred_element_type=jnp.float32)
                wo_h = wo_ref[h * DH:(h + 1) * DH, :].astype(bf16)
                acc = acc + jnp.dot(ctx.astype(bf16), wo_h,
                                    preferred_element_type=jnp.float32)
            acc_ref[b * Sq:(b + 1) * Sq, :] = acc

        i = lax.axis_index("i")
        z = i // 8
        p = lax.rem(i, 8)
        y = p // 2
        xb_ = jnp.bitwise_xor(lax.rem(p, 2), lax.rem(y, 2))

        def ring_idx(xx, yy, zz):
            pp = 2 * yy + jnp.bitwise_xor(xx, lax.rem(yy, 2))
            return 8 * zz + pp

        partners = [
            ring_idx(jnp.bitwise_xor(xb_, 1), y, z),
            ring_idx(xb_, jnp.bitwise_xor(y, 1), z),
            ring_idx(xb_, y, jnp.bitwise_xor(z, 1)),
            ring_idx(xb_, jnp.bitwise_xor(y, 2), z),
            ring_idx(xb_, y, jnp.bitwise_xor(z, 2)),
        ]
        bits = [
            lax.rem(xb_, 2),
            lax.rem(y, 2),
            lax.rem(z, 2),
            lax.rem(y // 2, 2),
            lax.rem(z // 2, 2),
        ]

        barrier_sem = pltpu.get_barrier_semaphore()
        for pk in partners:
            pl.semaphore_signal(
                barrier_sem, inc=1,
                device_id=(pk,), device_id_type=pl.DeviceIdType.MESH,
            )
        pl.semaphore_wait(barrier_sem, len(partners))

        seg_start = i * 0
        for k in range(5):
            half = rs_sizes[k]
            b = bits[k]
            send_off = seg_start + (1 - b) * half
            keep_off = seg_start + b * half
            send_stage[pl.ds(0, half), :] = acc_ref[
                pl.ds(send_off, half), :
            ].astype(bf16)
            rdma = pltpu.make_async_remote_copy(
                src_ref=send_stage.at[pl.ds(0, half)],
                dst_ref=rs_recv.at[pl.ds(rs_offs[k], half)],
                send_sem=send_sems.at[k],
                recv_sem=recv_sems.at[k],
                device_id=(partners[k],),
                device_id_type=pl.DeviceIdType.MESH,
            )
            rdma.start()
            rdma.wait()
            acc_ref[pl.ds(keep_off, half), :] = (
                acc_ref[pl.ds(keep_off, half), :]
                + rs_recv[pl.ds(rs_offs[k], half), :].astype(jnp.float32)
            )
            seg_start = keep_off

        own = m // N_DEV
        ag_buf[pl.ds(seg_start, own), :] = acc_ref[
            pl.ds(seg_start, own), :
        ].astype(bf16)
        own_start = seg_start
        for j in range(5):
            r = 4 - j
            b = bits[r]
            rdma = pltpu.make_async_remote_copy(
                src_ref=ag_buf.at[pl.ds(own_start, own)],
                dst_ref=ag_buf.at[pl.ds(own_start, own)],
                send_sem=send_sems.at[5 + j],
                recv_sem=recv_sems.at[5 + j],
                device_id=(partners[r],),
                device_id_type=pl.DeviceIdType.MESH,
            )
            rdma.start()
            rdma.wait()
            own_start = own_start - b * own
            own = own * 2

        out_ref[:, :] = ag_buf[:, :].astype(jnp.float32)

    return pl.pallas_call(
        body,
        out_shape=jax.ShapeDtypeStruct((m, n), jnp.float32),
        in_specs=[pl.BlockSpec(memory_space=pltpu.VMEM)] * 5,
        out_specs=pl.BlockSpec(memory_space=pltpu.VMEM),
        scratch_shapes=[
            pltpu.VMEM((m, n), jnp.float32),
            pltpu.VMEM((m // 2, n), bf16),
            pltpu.VMEM((rs_total, n), bf16),
            pltpu.VMEM((m, n), bf16),
            pltpu.SemaphoreType.DMA((10,)),
            pltpu.SemaphoreType.DMA((10,)),
        ],
        compiler_params=pltpu.CompilerParams(collective_id=0),
    )(x, Wq, K, V, Wo)


def kernel(x, Wq, K_ext, V_ext, Wo):
    my = lax.axis_index("i")
    B, Sq, E = x.shape
    h_per = Wq.shape[1] // DH

    K = lax.dynamic_slice_in_dim(K_ext, my * h_per, h_per, axis=2)
    V = lax.dynamic_slice_in_dim(V_ext, my * h_per, h_per, axis=2)
    K = K.astype(jnp.bfloat16)
    V = V.astype(jnp.bfloat16)

    out = _fused(x, Wq, K, V, Wo)
    return out.reshape(B, Sq, E)
